# Initial kernel scaffold; baseline (speedup 1.0000x reference)
import jax
import jax.numpy as jnp
from jax.experimental import pallas as pl


def kernel(x, edge_index, batch, descriptors, W0, b0, W1, b1, W2, b2, W3, b3, Wd, bd, Wlin, blin):
    raise NotImplementedError("write your pallas kernel here")


# speedup vs baseline: 7.2601x; 7.2601x over previous
import jax
import jax.numpy as jnp
from jax import lax
from jax.experimental import pallas as pl
from jax.experimental.pallas import tpu as pltpu
from jax.experimental.pallas import tpu_sc as plsc

N = 10000
E = 160000
CH0 = 256
H = 512
G = 64
DESC = 128

CK = 128
NCK = H // CK
NT = 16
NC = 2
EPT = E // NT
B = 128
NB = (EPT + B - 1) // B
EPP = NB * B
RPT = 632
RPT_LAST = N - (NT - 1) * RPT
ACC_ROWS = NT * RPT

_f32 = jnp.float32


def _copy_rows(s, src_ref, dst_ref):
  r0 = pl.multiple_of(s * RPT, 8)

  @pl.when(s < NT - 1)
  def _():
    pltpu.sync_copy(src_ref.at[pl.ds(r0, RPT)], dst_ref.at[pl.ds(r0, RPT)])

  @pl.when(s == NT - 1)
  def _():
    base = (NT - 1) * RPT
    pltpu.sync_copy(src_ref.at[pl.ds(base, RPT_LAST)],
                    dst_ref.at[pl.ds(base, RPT_LAST)])


def _sc_mesh():
  return plsc.VectorSubcoreMesh(
      core_axis_name="c", subcore_axis_name="s",
      num_cores=NC, num_subcores=NT)


def _agg_body(y0, y1, y2, y3, srcp, dstp, o0, o1, o2, o3,
              src_v, dst_v, buf, sem, acc):
  c = lax.axis_index("c")
  s = lax.axis_index("s")
  pltpu.sync_copy(srcp.at[s], src_v)
  pltpu.sync_copy(dstp.at[s], dst_v)

  def process(y_ref, o_ref):
    _copy_rows(s, y_ref, acc)
    plsc.subcore_barrier()

    def body(j, carry):
      pltpu.async_copy(y_ref.at[src_v.at[j]], buf, sem).wait()
      pltpu.sync_copy(buf, acc.at[dst_v.at[j]], add=True)
      return carry

    lax.fori_loop(0, NB, body, 0)
    plsc.subcore_barrier()
    _copy_rows(s, acc, o_ref)
    plsc.subcore_barrier()

  @pl.when(c == 0)
  def _():
    process(y0, o0)
    process(y1, o1)

  @pl.when(c == 1)
  def _():
    process(y2, o2)
    process(y3, o3)


def _agg_call(y_chunks, srcp, dstp):
  fn = pl.kernel(
      _agg_body,
      out_type=[jax.ShapeDtypeStruct((N, CK), _f32)] * NCK,
      mesh=_sc_mesh(),
      scratch_types=[
          pltpu.VMEM((NB, B), jnp.int32),
          pltpu.VMEM((NB, B), jnp.int32),
          pltpu.VMEM((B, CK), _f32),
          pltpu.SemaphoreType.DMA,
          pltpu.VMEM_SHARED((ACC_ROWS, CK), _f32),
      ],
  )
  return fn(*y_chunks, srcp, dstp)


def _deg_body(dstp, ones_h, zeros_h, degf, dst_v, buf, acc):
  c = lax.axis_index("c")
  s = lax.axis_index("s")

  @pl.when(c == 0)
  def _():
    pltpu.sync_copy(dstp.at[s], dst_v)
    pltpu.sync_copy(ones_h, buf)
    z0 = pl.multiple_of(s * RPT, 8)
    pltpu.sync_copy(zeros_h.at[pl.ds(z0, RPT)], acc.at[pl.ds(z0, RPT)])
    plsc.subcore_barrier()

    def body(j, carry):
      pltpu.sync_copy(buf, acc.at[dst_v.at[j]], add=True)
      return carry

    lax.fori_loop(0, NB, body, 0)
    plsc.subcore_barrier()
    _copy_rows(s, acc, degf)


def _deg_call(dstp, ones_h, zeros_h):
  fn = pl.kernel(
      _deg_body,
      out_type=jax.ShapeDtypeStruct((N, CK), _f32),
      mesh=_sc_mesh(),
      scratch_types=[
          pltpu.VMEM((NB, B), jnp.int32),
          pltpu.VMEM((B, CK), _f32),
          pltpu.VMEM_SHARED((ACC_ROWS, CK), _f32),
      ],
  )
  return fn(dstp, ones_h, zeros_h)


R0 = 1000


def _tc0_body(x_ref, w_ref, deg_ref, *y_refs):
  dinv = lax.rsqrt(deg_ref[:, 0:1] + 1.0)
  y = jnp.dot(x_ref[...] * dinv, w_ref[...], preferred_element_type=_f32)
  for k, yr in enumerate(y_refs):
    yr[...] = y[:, k * CK:(k + 1) * CK]


def _tc0(x, W, degf):
  return pl.pallas_call(
      _tc0_body,
      grid=(N // R0,),
      in_specs=[
          pl.BlockSpec((R0, CH0), lambda i: (i, 0)),
          pl.BlockSpec((CH0, H), lambda i: (0, 0)),
          pl.BlockSpec((R0, CK), lambda i: (i, 0)),
      ],
      out_specs=[pl.BlockSpec((R0, CK), lambda i: (i, 0))] * NCK,
      out_shape=[jax.ShapeDtypeStruct((N, CK), _f32)] * NCK,
  )(x, W, degf)


def _tcmid_body(a0, a1, a2, a3, w_ref, deg_ref, b_ref, *y_refs):
  dinv = lax.rsqrt(deg_ref[:, 0:1] + 1.0)
  h = jnp.concatenate([a0[...], a1[...], a2[...], a3[...]], axis=1)
  h = jnp.maximum(h * dinv + b_ref[...], 0.0)
  y = jnp.dot(h * dinv, w_ref[...], preferred_element_type=_f32)
  for k, yr in enumerate(y_refs):
    yr[...] = y[:, k * CK:(k + 1) * CK]


def _tcmid(acc, W, b, degf):
  return pl.pallas_call(
      _tcmid_body,
      grid=(N // R0,),
      in_specs=[pl.BlockSpec((R0, CK), lambda i: (i, 0))] * NCK + [
          pl.BlockSpec((H, H), lambda i: (0, 0)),
          pl.BlockSpec((R0, CK), lambda i: (i, 0)),
          pl.BlockSpec((1, H), lambda i: (0, 0)),
      ],
      out_specs=[pl.BlockSpec((R0, CK), lambda i: (i, 0))] * NCK,
      out_shape=[jax.ShapeDtypeStruct((N, CK), _f32)] * NCK,
  )(*acc, W, degf, b.reshape(1, H))


RF = 400
NGF = N // RF


def _fin_body(a0, a1, a2, a3, deg_ref, b_ref, batch_ref, desc_ref,
              wd_ref, bd_ref, wl_ref, bl_ref, out_ref, sums, counts):
  i = pl.program_id(0)

  @pl.when(i == 0)
  def _():
    sums[...] = jnp.zeros_like(sums)
    counts[...] = jnp.zeros_like(counts)

  dinv = lax.rsqrt(deg_ref[:, 0:1] + 1.0)
  h = jnp.concatenate([a0[...], a1[...], a2[...], a3[...]], axis=1)
  h = jnp.maximum(h * dinv + b_ref[...], 0.0)
  gids = lax.broadcasted_iota(jnp.int32, (RF, G), 1)
  P = (batch_ref[...] == gids).astype(_f32)
  sums[...] += lax.dot_general(P, h, (((0,), (0,)), ((), ())),
                               preferred_element_type=_f32)
  counts[...] += lax.dot_general(P, jnp.ones((RF, 1), _f32),
                                 (((0,), (0,)), ((), ())),
                                 preferred_element_type=_f32)

  @pl.when(i == NGF - 1)
  def _():
    gm = sums[...] / jnp.maximum(counts[...], 1.0)
    de = jnp.maximum(
        jnp.dot(desc_ref[...], wd_ref[...], preferred_element_type=_f32)
        + bd_ref[...], 0.0)
    z = jnp.concatenate([gm, de], axis=1)
    logit = jnp.dot(z, wl_ref[...], preferred_element_type=_f32) + bl_ref[...]
    out_ref[...] = jax.nn.sigmoid(logit)


def _tcfinal(acc, b, degf, batch, descriptors, Wd, bd, Wlin, blin):
  return pl.pallas_call(
      _fin_body,
      grid=(NGF,),
      in_specs=[pl.BlockSpec((RF, CK), lambda i: (i, 0))] * NCK + [
          pl.BlockSpec((RF, CK), lambda i: (i, 0)),
          pl.BlockSpec((1, H), lambda i: (0, 0)),
          pl.BlockSpec((RF, 1), lambda i: (i, 0)),
          pl.BlockSpec((G, DESC), lambda i: (0, 0)),
          pl.BlockSpec((DESC, H), lambda i: (0, 0)),
          pl.BlockSpec((1, H), lambda i: (0, 0)),
          pl.BlockSpec((2 * H, 1), lambda i: (0, 0)),
          pl.BlockSpec((1, 1), lambda i: (0, 0)),
      ],
      out_specs=pl.BlockSpec((G, 1), lambda i: (0, 0)),
      out_shape=jax.ShapeDtypeStruct((G, 1), _f32),
      scratch_shapes=[
          pltpu.VMEM((G, H), _f32),
          pltpu.VMEM((G, 1), _f32),
      ],
  )(*acc, degf, b.reshape(1, H), batch.reshape(N, 1), descriptors,
    Wd, bd.reshape(1, H), Wlin, blin.reshape(1, 1))


def kernel(x, edge_index, batch, descriptors,
           W0, b0, W1, b1, W2, b2, W3, b3, Wd, bd, Wlin, blin):
  src = edge_index[0].reshape(NT, EPT)
  dst = edge_index[1].reshape(NT, EPT)
  pad = EPP - EPT
  srcp = jnp.pad(src, ((0, 0), (0, pad)), constant_values=0).reshape(NT, NB, B)
  dstp = jnp.pad(dst, ((0, 0), (0, pad)), constant_values=N).reshape(NT, NB, B)
  ones_h = jnp.ones((B, CK), _f32)
  zeros_h = jnp.zeros((ACC_ROWS, CK), _f32)

  degf = _deg_call(dstp, ones_h, zeros_h)
  y = _tc0(x, W0, degf)
  bs = [b0, b1, b2, b3]
  Ws = [W1, W2, W3]
  for l in range(3):
    acc = _agg_call(y, srcp, dstp)
    y = _tcmid(acc, Ws[l], bs[l], degf)
  acc = _agg_call(y, srcp, dstp)
  out = _tcfinal(acc, bs[3], degf, batch, descriptors, Wd, bd, Wlin, blin)
  return out.reshape(-1)
